# speedup vs baseline: 5.1354x; 5.1354x over previous
import functools

import jax
import jax.numpy as jnp
from jax import lax
from jax.experimental import pallas as pl
from jax.experimental.pallas import tpu as pltpu
from jax.experimental.pallas import tpu_sc as plsc

B = 16384
D = 768
C = 5
NC = 2
NS = 16
NW = NC * NS
BPW = B // NW
CHUNK = 64
NCHUNK = BPW // CHUNK


def _gather_sc(user_ids, item_ids, user_table, item_table):
    mesh = plsc.VectorSubcoreMesh(core_axis_name="c", subcore_axis_name="s")

    @functools.partial(
        pl.kernel,
        mesh=mesh,
        out_type=[
            jax.ShapeDtypeStruct((B, D), jnp.float32),
            jax.ShapeDtypeStruct((B, D), jnp.float32),
        ],
        scratch_types=[
            pltpu.VMEM((BPW,), jnp.int32),
            pltpu.VMEM((BPW,), jnp.int32),
            pltpu.VMEM((CHUNK, D), jnp.float32),
            pltpu.VMEM((CHUNK, D), jnp.float32),
            pltpu.SemaphoreType.DMA,
            pltpu.SemaphoreType.DMA,
        ],
    )
    def k(uid_hbm, iid_hbm, utab_hbm, itab_hbm, uout_hbm, iout_hbm,
          uidx, iidx, ubuf, ibuf, sem_u, sem_i):
        wid = lax.axis_index("s") * NC + lax.axis_index("c")
        base = wid * BPW
        pltpu.sync_copy(uid_hbm.at[pl.ds(base, BPW)], uidx)
        pltpu.sync_copy(iid_hbm.at[pl.ds(base, BPW)], iidx)
        for c in range(NCHUNK):
            off = c * CHUNK
            cu = pltpu.async_copy(utab_hbm.at[uidx.at[pl.ds(off, CHUNK)]],
                                  ubuf, sem_u)
            ci = pltpu.async_copy(itab_hbm.at[iidx.at[pl.ds(off, CHUNK)]],
                                  ibuf, sem_i)
            cu.wait()
            ci.wait()
            pltpu.sync_copy(ubuf, uout_hbm.at[pl.ds(base + off, CHUNK)])
            pltpu.sync_copy(ibuf, iout_hbm.at[pl.ds(base + off, CHUNK)])

    return k(user_ids, item_ids, user_table, item_table)


BM = 1024


def _mm_body(u_ref, i_ref, wu_ref, wi_ref, b_ref, o_ref):
    acc = jnp.dot(u_ref[...], wu_ref[...], preferred_element_type=jnp.float32)
    acc = acc + jnp.dot(i_ref[...], wi_ref[...],
                        preferred_element_type=jnp.float32)
    o_ref[...] = acc + b_ref[...]


def _mm_tc(u_rep, i_rep, wu, wi, b2d):
    return pl.pallas_call(
        _mm_body,
        grid=(B // BM,),
        in_specs=[
            pl.BlockSpec((BM, D), lambda m: (m, 0)),
            pl.BlockSpec((BM, D), lambda m: (m, 0)),
            pl.BlockSpec((D, C), lambda m: (0, 0)),
            pl.BlockSpec((D, C), lambda m: (0, 0)),
            pl.BlockSpec((1, C), lambda m: (0, 0)),
        ],
        out_specs=pl.BlockSpec((BM, C), lambda m: (m, 0)),
        out_shape=jax.ShapeDtypeStruct((B, C), jnp.float32),
    )(u_rep, i_rep, wu, wi, b2d)


def kernel(user_ids, item_ids, user_table, item_table, W, b):
    uids = user_ids.astype(jnp.int32)
    iids = item_ids.astype(jnp.int32)
    u_rep, i_rep = _gather_sc(uids, iids, user_table, item_table)
    return _mm_tc(u_rep, i_rep, W[:D], W[D:], b.reshape(1, C))


# speedup vs baseline: 5.3236x; 1.0367x over previous
import functools

import jax
import jax.numpy as jnp
from jax import lax
from jax.experimental import pallas as pl
from jax.experimental.pallas import tpu as pltpu
from jax.experimental.pallas import tpu_sc as plsc

B = 16384
D = 768
C = 5
NC = 2
NS = 16
NW = NC * NS
BPW = B // NW
CHUNK = 32
NCHUNK = BPW // CHUNK


def _gather_sc(user_ids, item_ids, user_table, item_table):
    mesh = plsc.VectorSubcoreMesh(core_axis_name="c", subcore_axis_name="s")

    @functools.partial(
        pl.kernel,
        mesh=mesh,
        out_type=[
            jax.ShapeDtypeStruct((B, D), jnp.float32),
            jax.ShapeDtypeStruct((B, D), jnp.float32),
        ],
        scratch_types=[
            pltpu.VMEM((BPW,), jnp.int32),
            pltpu.VMEM((BPW,), jnp.int32),
            pltpu.VMEM((2, CHUNK, D), jnp.float32),
            pltpu.VMEM((2, CHUNK, D), jnp.float32),
            [pltpu.SemaphoreType.DMA] * 2,
            [pltpu.SemaphoreType.DMA] * 2,
            [pltpu.SemaphoreType.DMA] * 2,
            [pltpu.SemaphoreType.DMA] * 2,
        ],
    )
    def k(uid_hbm, iid_hbm, utab_hbm, itab_hbm, uout_hbm, iout_hbm,
          uidx, iidx, ubuf, ibuf, gsem_u, gsem_i, osem_u, osem_i):
        wid = lax.axis_index("s") * NC + lax.axis_index("c")
        base = wid * BPW
        pltpu.sync_copy(uid_hbm.at[pl.ds(base, BPW)], uidx)
        pltpu.sync_copy(iid_hbm.at[pl.ds(base, BPW)], iidx)

        def start_gather(c):
            bsl = c % 2
            off = c * CHUNK
            pltpu.async_copy(utab_hbm.at[uidx.at[pl.ds(off, CHUNK)]],
                             ubuf.at[bsl], gsem_u[bsl])
            pltpu.async_copy(itab_hbm.at[iidx.at[pl.ds(off, CHUNK)]],
                             ibuf.at[bsl], gsem_i[bsl])

        def wait_gather(c):
            bsl = c % 2
            pltpu.make_async_copy(utab_hbm.at[uidx.at[pl.ds(0, CHUNK)]],
                                  ubuf.at[bsl], gsem_u[bsl]).wait()
            pltpu.make_async_copy(itab_hbm.at[iidx.at[pl.ds(0, CHUNK)]],
                                  ibuf.at[bsl], gsem_i[bsl]).wait()

        def start_out(c):
            bsl = c % 2
            off = c * CHUNK
            pltpu.async_copy(ubuf.at[bsl], uout_hbm.at[pl.ds(base + off, CHUNK)],
                             osem_u[bsl])
            pltpu.async_copy(ibuf.at[bsl], iout_hbm.at[pl.ds(base + off, CHUNK)],
                             osem_i[bsl])

        def wait_out(c):
            bsl = c % 2
            off = c * CHUNK
            pltpu.make_async_copy(ubuf.at[bsl],
                                  uout_hbm.at[pl.ds(base + off, CHUNK)],
                                  osem_u[bsl]).wait()
            pltpu.make_async_copy(ibuf.at[bsl],
                                  iout_hbm.at[pl.ds(base + off, CHUNK)],
                                  osem_i[bsl]).wait()

        for c in range(NCHUNK):
            if c >= 2:
                wait_out(c - 2)
            start_gather(c)
            if c >= 1:
                wait_gather(c - 1)
                start_out(c - 1)
        wait_gather(NCHUNK - 1)
        start_out(NCHUNK - 1)
        wait_out(NCHUNK - 2)
        wait_out(NCHUNK - 1)

    return k(user_ids, item_ids, user_table, item_table)


BM = 1024


def _mm_body(u_ref, i_ref, wu_ref, wi_ref, b_ref, o_ref):
    acc = jnp.dot(u_ref[...], wu_ref[...], preferred_element_type=jnp.float32)
    acc = acc + jnp.dot(i_ref[...], wi_ref[...],
                        preferred_element_type=jnp.float32)
    o_ref[...] = acc + b_ref[...]


def _mm_tc(u_rep, i_rep, wu, wi, b2d):
    return pl.pallas_call(
        _mm_body,
        grid=(B // BM,),
        in_specs=[
            pl.BlockSpec((BM, D), lambda m: (m, 0)),
            pl.BlockSpec((BM, D), lambda m: (m, 0)),
            pl.BlockSpec((D, C), lambda m: (0, 0)),
            pl.BlockSpec((D, C), lambda m: (0, 0)),
            pl.BlockSpec((1, C), lambda m: (0, 0)),
        ],
        out_specs=pl.BlockSpec((BM, C), lambda m: (m, 0)),
        out_shape=jax.ShapeDtypeStruct((B, C), jnp.float32),
    )(u_rep, i_rep, wu, wi, b2d)


def kernel(user_ids, item_ids, user_table, item_table, W, b):
    uids = user_ids.astype(jnp.int32)
    iids = item_ids.astype(jnp.int32)
    u_rep, i_rep = _gather_sc(uids, iids, user_table, item_table)
    return _mm_tc(u_rep, i_rep, W[:D], W[D:], b.reshape(1, C))
